# baseline (speedup 1.0000x reference)
import functools

import jax
import jax.numpy as jnp
import numpy as np
from jax import lax
from jax.experimental import pallas as pl

_STAGES = [64, 128]
_NUM_HEADS = [8, 16]
_K_MHA = [27, 27]
_POOL_K = [27, 27]
_POOL_FACTOR = [0.25, 0.25]


def _knn_body(K, M, a_ref, b_ref, idx_ref):
    a = a_ref[0]
    b = b_ref[0]
    an = jnp.sum(a * a, axis=-1)
    bn = jnp.sum(b * b, axis=-1)
    d = (an[:, None] + bn[None, :]
         - 2.0 * lax.dot_general(a, b, (((1,), (1,)), ((), ())),
                                 preferred_element_type=jnp.float32))
    col = lax.broadcasted_iota(jnp.int32, d.shape, 1)
    R = d.shape[0]
    Kp = idx_ref.shape[2]
    kcol = lax.broadcasted_iota(jnp.int32, (R, Kp), 1)
    inf = jnp.float32(jnp.inf)

    def step(t, carry):
        dcur, acc = carry
        v = jnp.min(dcur, axis=-1)
        is_min = dcur == v[:, None]
        i = jnp.min(jnp.where(is_min, col, M), axis=-1)
        acc = jnp.where(kcol == t, i[:, None], acc)
        dcur = jnp.where(col == i[:, None], inf, dcur)
        return dcur, acc

    acc0 = jnp.zeros((R, Kp), jnp.int32)
    _, acc = lax.fori_loop(0, K, step, (d, acc0), unroll=False)
    idx_ref[0] = acc


def _knn_topk(a, b, K, block_r=256):
    B, N, _ = a.shape
    M = b.shape[1]
    block_r = min(block_r, N)
    return pl.pallas_call(
        functools.partial(_knn_body, K, M),
        grid=(B, N // block_r),
        in_specs=[
            pl.BlockSpec((1, block_r, 3), lambda bb, rr: (bb, rr, 0)),
            pl.BlockSpec((1, M, 3), lambda bb, rr: (bb, 0, 0)),
        ],
        out_specs=pl.BlockSpec((1, block_r, K), lambda bb, rr: (bb, rr, 0)),
        out_shape=jax.ShapeDtypeStruct((B, N, K), jnp.int32),
    )(a, b)


def _topk_scores_body(V, n, ib, scol_ref, srow_ref, out_ref):
    srow = srow_ref[0]
    jcol = lax.broadcasted_iota(jnp.int32, (ib, V), 1)
    riota = lax.broadcasted_iota(jnp.int32, (1, n), 1)
    out = jnp.zeros((1, n), jnp.int32)
    for blk in range(V // ib):
        i0 = blk * ib
        si = scol_ref[0, i0:i0 + ib, :]
        ii = i0 + lax.broadcasted_iota(jnp.int32, (ib, 1), 0)
        gt = (srow > si).astype(jnp.int32)
        eqlt = ((srow == si) & (jcol < ii)).astype(jnp.int32)
        rank = jnp.sum(gt + eqlt, axis=1, keepdims=True)
        hit = rank == riota
        out = out + jnp.sum(jnp.where(hit, ii, 0), axis=0, keepdims=True)
    out_ref[0] = out


def _topk_scores(scores, n, ib=256):
    B, V = scores.shape
    return pl.pallas_call(
        functools.partial(_topk_scores_body, V, n, ib),
        grid=(B,),
        in_specs=[
            pl.BlockSpec((1, V, 1), lambda bb: (bb, 0, 0)),
            pl.BlockSpec((1, 1, V), lambda bb: (bb, 0, 0)),
        ],
        out_specs=pl.BlockSpec((1, 1, n), lambda bb: (bb, 0, 0)),
        out_shape=jax.ShapeDtypeStruct((B, 1, n), jnp.int32),
    )(scores[:, :, None], scores[:, None, :]).reshape(B, n)


def _matmul_kernel(x_ref, w_ref, o_ref):
    o_ref[...] = jnp.dot(x_ref[...], w_ref[...],
                         preferred_element_type=jnp.float32)


def _pl_matmul(x, w):
    B, N, d = x.shape
    e = w.shape[1]
    out = pl.pallas_call(
        _matmul_kernel,
        out_shape=jax.ShapeDtypeStruct((B * N, e), jnp.float32),
    )(x.reshape(B * N, d), w)
    return out.reshape(B, N, e)


def _pairwise_sqdist(a, b):
    return (jnp.sum(a * a, -1)[:, :, None] + jnp.sum(b * b, -1)[:, None, :]
            - 2.0 * jnp.einsum('bnd,bmd->bnm', a, b))


def _gather_rows(x, idx):
    return jax.vmap(lambda xb, ib: xb[ib])(x, idx)


def _knn_idx_xla(a, b, K):
    d = _pairwise_sqdist(a, b)
    _, idx = jax.lax.top_k(-d, K)
    return idx


def _mha_knn_v(x, x_v, p, num_heads, K):
    Bb, Vv, d = x.shape
    idx = _knn_topk(x_v, x_v, K)
    k_feat = _gather_rows(x, idx)
    q = x @ p['Wq'] + p['bq']
    k = k_feat @ p['Wk'] + p['bk']
    v = k_feat @ p['Wv'] + p['bv']
    dh = d // num_heads
    q = q.reshape(Bb, Vv, num_heads, dh)
    k = k.reshape(Bb, Vv, K, num_heads, dh)
    v = v.reshape(Bb, Vv, K, num_heads, dh)
    attn = jnp.einsum('bvhd,bvkhd->bvhk', q, k) / np.sqrt(dh)
    attn = jax.nn.softmax(attn, axis=-1)
    out = jnp.einsum('bvhk,bvkhd->bvhd', attn, v).reshape(Bb, Vv, d)
    return out @ p['Wo'] + p['bo']


def _attention_pooling_v(x, x_v, p, K, pooling_factor):
    Bb, Vv, d = x.shape
    h = jax.nn.relu(x @ p['W1'] + p['b1'])
    s = jax.nn.sigmoid(h @ p['W2'] + p['b2'])
    scores = s[..., 0]
    n_pool = int(Vv * pooling_factor)
    pool_idx = _topk_scores(scores, n_pool)
    x_v_next = _gather_rows(x_v, pool_idx)
    nidx = _knn_topk(x_v_next, x_v, K) if Vv == 4096 else _knn_idx_xla(x_v_next, x_v, K)
    x_knn = _gather_rows(x * s, nidx)
    s_knn = _gather_rows(scores[..., None], nidx)[..., 0]
    w = jax.nn.softmax(s_knn, axis=-1)
    x_pooled = jnp.sum(w[..., None] * x_knn, axis=2)
    unpool_idx = jnp.argmin(_pairwise_sqdist(x_v, x_v_next), axis=-1)
    return x_pooled, x_v_next, s, pool_idx, unpool_idx


def kernel(x, x_v, params):
    x = x @ params['W_emb']
    unpooling = []
    for i in range(len(_STAGES)):
        p = params['stage%d' % i]
        x = _mha_knn_v(x, x_v, p['mha'], _NUM_HEADS[i], _K_MHA[i]) + x
        x_p, x_v_next, x_s, pool_idx, unpool_idx = _attention_pooling_v(
            x, x_v, p['pool'], _POOL_K[i], _POOL_FACTOR[i])
        unpooling.insert(0, (x_v, unpool_idx, x_s))
        x_v = x_v_next
        if i == len(_STAGES) - 1:
            x = _pl_matmul(x_p, p['Wout'])
        else:
            x = x_p @ p['Wout']
    return (x, unpooling[0][1], unpooling[1][1])


# speedup vs baseline: 1.1107x; 1.1107x over previous
import functools

import jax
import jax.numpy as jnp
import numpy as np
from jax import lax
from jax.experimental import pallas as pl

_STAGES = [64, 128]
_NUM_HEADS = [8, 16]
_K_MHA = [27, 27]
_POOL_K = [27, 27]
_POOL_FACTOR = [0.25, 0.25]


def _knn_body(K, M, a_ref, b_ref, idx_ref):
    a = a_ref[0]
    b = b_ref[0]
    an = jnp.sum(a * a, axis=-1)
    bn = jnp.sum(b * b, axis=-1)
    d = (an[:, None] + bn[None, :]
         - 2.0 * lax.dot_general(a, b, (((1,), (1,)), ((), ())),
                                 preferred_element_type=jnp.float32))
    col = lax.broadcasted_iota(jnp.int32, d.shape, 1)
    R = d.shape[0]
    Kp = idx_ref.shape[2]
    kcol = lax.broadcasted_iota(jnp.int32, (R, Kp), 1)
    inf = jnp.float32(jnp.inf)

    def step(t, carry):
        dcur, acc = carry
        v = jnp.min(dcur, axis=-1)
        is_min = dcur == v[:, None]
        i = jnp.min(jnp.where(is_min, col, M), axis=-1)
        acc = jnp.where(kcol == t, i[:, None], acc)
        dcur = jnp.where(col == i[:, None], inf, dcur)
        return dcur, acc

    acc0 = jnp.zeros((R, Kp), jnp.int32)
    _, acc = lax.fori_loop(0, K, step, (d, acc0), unroll=True)
    idx_ref[0] = acc


def _knn_topk(a, b, K, block_r=512):
    B, N, _ = a.shape
    M = b.shape[1]
    block_r = min(block_r, N)
    return pl.pallas_call(
        functools.partial(_knn_body, K, M),
        grid=(B, N // block_r),
        in_specs=[
            pl.BlockSpec((1, block_r, 3), lambda bb, rr: (bb, rr, 0)),
            pl.BlockSpec((1, M, 3), lambda bb, rr: (bb, 0, 0)),
        ],
        out_specs=pl.BlockSpec((1, block_r, K), lambda bb, rr: (bb, rr, 0)),
        out_shape=jax.ShapeDtypeStruct((B, N, K), jnp.int32),
    )(a, b)


def _topk_scores_body(V, n, ib, scol_ref, srow_ref, out_ref):
    srow = srow_ref[0]
    jcol = lax.broadcasted_iota(jnp.int32, (ib, V), 1)
    riota = lax.broadcasted_iota(jnp.int32, (1, n), 1)
    out = jnp.zeros((1, n), jnp.int32)
    for blk in range(V // ib):
        i0 = blk * ib
        si = scol_ref[0, i0:i0 + ib, :]
        ii = i0 + lax.broadcasted_iota(jnp.int32, (ib, 1), 0)
        gt = (srow > si).astype(jnp.int32)
        eqlt = ((srow == si) & (jcol < ii)).astype(jnp.int32)
        rank = jnp.sum(gt + eqlt, axis=1, keepdims=True)
        hit = rank == riota
        out = out + jnp.sum(jnp.where(hit, ii, 0), axis=0, keepdims=True)
    out_ref[0] = out


def _topk_scores(scores, n, ib=256):
    B, V = scores.shape
    return pl.pallas_call(
        functools.partial(_topk_scores_body, V, n, ib),
        grid=(B,),
        in_specs=[
            pl.BlockSpec((1, V, 1), lambda bb: (bb, 0, 0)),
            pl.BlockSpec((1, 1, V), lambda bb: (bb, 0, 0)),
        ],
        out_specs=pl.BlockSpec((1, 1, n), lambda bb: (bb, 0, 0)),
        out_shape=jax.ShapeDtypeStruct((B, 1, n), jnp.int32),
    )(scores[:, :, None], scores[:, None, :]).reshape(B, n)


def _matmul_kernel(x_ref, w_ref, o_ref):
    o_ref[...] = jnp.dot(x_ref[...], w_ref[...],
                         preferred_element_type=jnp.float32)


def _pl_matmul(x, w):
    B, N, d = x.shape
    e = w.shape[1]
    out = pl.pallas_call(
        _matmul_kernel,
        out_shape=jax.ShapeDtypeStruct((B * N, e), jnp.float32),
    )(x.reshape(B * N, d), w)
    return out.reshape(B, N, e)


def _pairwise_sqdist(a, b):
    return (jnp.sum(a * a, -1)[:, :, None] + jnp.sum(b * b, -1)[:, None, :]
            - 2.0 * jnp.einsum('bnd,bmd->bnm', a, b))


def _gather_rows(x, idx):
    return jax.vmap(lambda xb, ib: xb[ib])(x, idx)


def _knn_idx_xla(a, b, K):
    d = _pairwise_sqdist(a, b)
    _, idx = jax.lax.top_k(-d, K)
    return idx


def _mha_knn_v(x, x_v, p, num_heads, K):
    Bb, Vv, d = x.shape
    idx = _knn_topk(x_v, x_v, K)
    k_feat = _gather_rows(x, idx)
    q = x @ p['Wq'] + p['bq']
    k = k_feat @ p['Wk'] + p['bk']
    v = k_feat @ p['Wv'] + p['bv']
    dh = d // num_heads
    q = q.reshape(Bb, Vv, num_heads, dh)
    k = k.reshape(Bb, Vv, K, num_heads, dh)
    v = v.reshape(Bb, Vv, K, num_heads, dh)
    attn = jnp.einsum('bvhd,bvkhd->bvhk', q, k) / np.sqrt(dh)
    attn = jax.nn.softmax(attn, axis=-1)
    out = jnp.einsum('bvhk,bvkhd->bvhd', attn, v).reshape(Bb, Vv, d)
    return out @ p['Wo'] + p['bo']


def _attention_pooling_v(x, x_v, p, K, pooling_factor):
    Bb, Vv, d = x.shape
    h = jax.nn.relu(x @ p['W1'] + p['b1'])
    s = jax.nn.sigmoid(h @ p['W2'] + p['b2'])
    scores = s[..., 0]
    n_pool = int(Vv * pooling_factor)
    pool_idx = _topk_scores(scores, n_pool)
    x_v_next = _gather_rows(x_v, pool_idx)
    nidx = _knn_topk(x_v_next, x_v, K) if Vv == 4096 else _knn_idx_xla(x_v_next, x_v, K)
    x_knn = _gather_rows(x * s, nidx)
    s_knn = _gather_rows(scores[..., None], nidx)[..., 0]
    w = jax.nn.softmax(s_knn, axis=-1)
    x_pooled = jnp.sum(w[..., None] * x_knn, axis=2)
    unpool_idx = jnp.argmin(_pairwise_sqdist(x_v, x_v_next), axis=-1)
    return x_pooled, x_v_next, s, pool_idx, unpool_idx


def kernel(x, x_v, params):
    x = x @ params['W_emb']
    unpooling = []
    for i in range(len(_STAGES)):
        p = params['stage%d' % i]
        x = _mha_knn_v(x, x_v, p['mha'], _NUM_HEADS[i], _K_MHA[i]) + x
        x_p, x_v_next, x_s, pool_idx, unpool_idx = _attention_pooling_v(
            x, x_v, p['pool'], _POOL_K[i], _POOL_FACTOR[i])
        unpooling.insert(0, (x_v, unpool_idx, x_s))
        x_v = x_v_next
        if i == len(_STAGES) - 1:
            x = _pl_matmul(x_p, p['Wout'])
        else:
            x = x_p @ p['Wout']
    return (x, unpooling[0][1], unpooling[1][1])


# speedup vs baseline: 1.1130x; 1.0021x over previous
import functools

import jax
import jax.numpy as jnp
import numpy as np
from jax import lax
from jax.experimental import pallas as pl

_STAGES = [64, 128]
_NUM_HEADS = [8, 16]
_K_MHA = [27, 27]
_POOL_K = [27, 27]
_POOL_FACTOR = [0.25, 0.25]


def _knn_body(K, M, a_ref, b_ref, idx_ref):
    a = a_ref[0]
    b = b_ref[0]
    an = jnp.sum(a * a, axis=-1)
    bn = jnp.sum(b * b, axis=-1)
    d = (an[:, None] + bn[None, :]
         - 2.0 * lax.dot_general(a, b, (((1,), (1,)), ((), ())),
                                 preferred_element_type=jnp.float32))
    col = lax.broadcasted_iota(jnp.int32, d.shape, 1)
    R = d.shape[0]
    Kp = idx_ref.shape[2]
    kcol = lax.broadcasted_iota(jnp.int32, (R, Kp), 1)
    inf = jnp.float32(jnp.inf)

    def step(t, carry):
        dcur, acc = carry
        v = jnp.min(dcur, axis=-1)
        is_min = dcur == v[:, None]
        i = jnp.min(jnp.where(is_min, col, M), axis=-1)
        acc = jnp.where(kcol == t, i[:, None], acc)
        dcur = jnp.where(col == i[:, None], inf, dcur)
        return dcur, acc

    acc0 = jnp.zeros((R, Kp), jnp.int32)
    _, acc = lax.fori_loop(0, K, step, (d, acc0), unroll=True)
    idx_ref[0] = acc


def _knn_topk(a, b, K, block_r=512):
    B, N, _ = a.shape
    M = b.shape[1]
    block_r = min(block_r, N)
    return pl.pallas_call(
        functools.partial(_knn_body, K, M),
        grid=(B, N // block_r),
        in_specs=[
            pl.BlockSpec((1, block_r, 3), lambda bb, rr: (bb, rr, 0)),
            pl.BlockSpec((1, M, 3), lambda bb, rr: (bb, 0, 0)),
        ],
        out_specs=pl.BlockSpec((1, block_r, K), lambda bb, rr: (bb, rr, 0)),
        out_shape=jax.ShapeDtypeStruct((B, N, K), jnp.int32),
    )(a, b)


def _argmin_body(M, a_ref, b_ref, idx_ref):
    a = a_ref[0]
    b = b_ref[0]
    an = jnp.sum(a * a, axis=-1)
    bn = jnp.sum(b * b, axis=-1)
    d = (an[:, None] + bn[None, :]
         - 2.0 * lax.dot_general(a, b, (((1,), (1,)), ((), ())),
                                 preferred_element_type=jnp.float32))
    col = lax.broadcasted_iota(jnp.int32, d.shape, 1)
    v = jnp.min(d, axis=-1)
    i = jnp.min(jnp.where(d == v[:, None], col, M), axis=-1)
    idx_ref[0] = i[:, None]


def _argmin_dist(a, b, block_r=512):
    B, N, _ = a.shape
    M = b.shape[1]
    block_r = min(block_r, N)
    out = pl.pallas_call(
        functools.partial(_argmin_body, M),
        grid=(B, N // block_r),
        in_specs=[
            pl.BlockSpec((1, block_r, 3), lambda bb, rr: (bb, rr, 0)),
            pl.BlockSpec((1, M, 3), lambda bb, rr: (bb, 0, 0)),
        ],
        out_specs=pl.BlockSpec((1, block_r, 1), lambda bb, rr: (bb, rr, 0)),
        out_shape=jax.ShapeDtypeStruct((B, N, 1), jnp.int32),
    )(a, b)
    return out.reshape(B, N)


def _topk_scores_body(V, n, ib, scol_ref, srow_ref, out_ref):
    srow = srow_ref[0]
    jcol = lax.broadcasted_iota(jnp.int32, (ib, V), 1)
    riota = lax.broadcasted_iota(jnp.int32, (1, n), 1)
    out = jnp.zeros((1, n), jnp.int32)
    for blk in range(V // ib):
        i0 = blk * ib
        si = scol_ref[0, i0:i0 + ib, :]
        ii = i0 + lax.broadcasted_iota(jnp.int32, (ib, 1), 0)
        gt = (srow > si).astype(jnp.int32)
        eqlt = ((srow == si) & (jcol < ii)).astype(jnp.int32)
        rank = jnp.sum(gt + eqlt, axis=1, keepdims=True)
        hit = rank == riota
        out = out + jnp.sum(jnp.where(hit, ii, 0), axis=0, keepdims=True)
    out_ref[0] = out


def _topk_scores(scores, n, ib=256):
    B, V = scores.shape
    return pl.pallas_call(
        functools.partial(_topk_scores_body, V, n, ib),
        grid=(B,),
        in_specs=[
            pl.BlockSpec((1, V, 1), lambda bb: (bb, 0, 0)),
            pl.BlockSpec((1, 1, V), lambda bb: (bb, 0, 0)),
        ],
        out_specs=pl.BlockSpec((1, 1, n), lambda bb: (bb, 0, 0)),
        out_shape=jax.ShapeDtypeStruct((B, 1, n), jnp.int32),
    )(scores[:, :, None], scores[:, None, :]).reshape(B, n)


def _matmul_kernel(x_ref, w_ref, o_ref):
    o_ref[...] = jnp.dot(x_ref[...], w_ref[...],
                         preferred_element_type=jnp.float32)


def _pl_matmul(x, w):
    B, N, d = x.shape
    e = w.shape[1]
    out = pl.pallas_call(
        _matmul_kernel,
        out_shape=jax.ShapeDtypeStruct((B * N, e), jnp.float32),
    )(x.reshape(B * N, d), w)
    return out.reshape(B, N, e)


def _pairwise_sqdist(a, b):
    return (jnp.sum(a * a, -1)[:, :, None] + jnp.sum(b * b, -1)[:, None, :]
            - 2.0 * jnp.einsum('bnd,bmd->bnm', a, b))


def _gather_rows(x, idx):
    return jax.vmap(lambda xb, ib: xb[ib])(x, idx)


def _knn_idx_xla(a, b, K):
    d = _pairwise_sqdist(a, b)
    _, idx = jax.lax.top_k(-d, K)
    return idx


def _mha_knn_v(x, x_v, p, num_heads, K):
    Bb, Vv, d = x.shape
    idx = _knn_topk(x_v, x_v, K)
    k_feat = _gather_rows(x, idx)
    q = x @ p['Wq'] + p['bq']
    k = k_feat @ p['Wk'] + p['bk']
    v = k_feat @ p['Wv'] + p['bv']
    dh = d // num_heads
    q = q.reshape(Bb, Vv, num_heads, dh)
    k = k.reshape(Bb, Vv, K, num_heads, dh)
    v = v.reshape(Bb, Vv, K, num_heads, dh)
    attn = jnp.einsum('bvhd,bvkhd->bvhk', q, k) / np.sqrt(dh)
    attn = jax.nn.softmax(attn, axis=-1)
    out = jnp.einsum('bvhk,bvkhd->bvhd', attn, v).reshape(Bb, Vv, d)
    return out @ p['Wo'] + p['bo']


def _attention_pooling_v(x, x_v, p, K, pooling_factor):
    Bb, Vv, d = x.shape
    h = jax.nn.relu(x @ p['W1'] + p['b1'])
    s = jax.nn.sigmoid(h @ p['W2'] + p['b2'])
    scores = s[..., 0]
    n_pool = int(Vv * pooling_factor)
    pool_idx = _topk_scores(scores, n_pool)
    x_v_next = _gather_rows(x_v, pool_idx)
    nidx = _knn_topk(x_v_next, x_v, K) if Vv == 4096 else _knn_idx_xla(x_v_next, x_v, K)
    x_knn = _gather_rows(x * s, nidx)
    s_knn = _gather_rows(scores[..., None], nidx)[..., 0]
    w = jax.nn.softmax(s_knn, axis=-1)
    x_pooled = jnp.sum(w[..., None] * x_knn, axis=2)
    if Vv == 4096:
        unpool_idx = _argmin_dist(x_v, x_v_next)
    else:
        unpool_idx = jnp.argmin(_pairwise_sqdist(x_v, x_v_next), axis=-1)
    return x_pooled, x_v_next, s, pool_idx, unpool_idx


def kernel(x, x_v, params):
    x = x @ params['W_emb']
    unpooling = []
    for i in range(len(_STAGES)):
        p = params['stage%d' % i]
        x = _mha_knn_v(x, x_v, p['mha'], _NUM_HEADS[i], _K_MHA[i]) + x
        x_p, x_v_next, x_s, pool_idx, unpool_idx = _attention_pooling_v(
            x, x_v, p['pool'], _POOL_K[i], _POOL_FACTOR[i])
        unpooling.insert(0, (x_v, unpool_idx, x_s))
        x_v = x_v_next
        if i == len(_STAGES) - 1:
            x = _pl_matmul(x_p, p['Wout'])
        else:
            x = x_p @ p['Wout']
    return (x, unpooling[0][1], unpooling[1][1])
